# Initial kernel scaffold; baseline (speedup 1.0000x reference)
import jax
import jax.numpy as jnp
from jax.experimental import pallas as pl


def kernel(x, weight):
    raise NotImplementedError("write your pallas kernel here")


# speedup vs baseline: 1.4620x; 1.4620x over previous
import functools

import jax
import jax.numpy as jnp
from jax import lax
from jax.experimental import pallas as pl
from jax.experimental.pallas import tpu as pltpu
from jax.experimental.pallas import tpu_sc as plsc


def _gather_sc(idx, weight, n, d):
    info = plsc.get_sparse_core_info()
    nc, ns = info.num_cores, info.num_subcores
    nw = nc * ns
    per_w = n // nw
    chunk = 1024
    n_chunks = per_w // chunk

    mesh = plsc.VectorSubcoreMesh(core_axis_name="c", subcore_axis_name="s")

    @functools.partial(
        pl.kernel,
        out_type=jax.ShapeDtypeStruct((n, d), jnp.float32),
        mesh=mesh,
        scratch_types=[
            pltpu.VMEM((chunk,), jnp.int32),
            pltpu.VMEM((chunk, d), jnp.float32),
            pltpu.SemaphoreType.DMA,
        ],
        compiler_params=pltpu.CompilerParams(use_tc_tiling_on_sc=False),
    )
    def k(idx_hbm, table_hbm, out_hbm, idx_v, rows_v, sem):
        wid = lax.axis_index("s") * nc + lax.axis_index("c")
        base = wid * per_w

        def body(j, carry):
            off = base + j * chunk
            pltpu.sync_copy(idx_hbm.at[pl.ds(off, chunk)], idx_v)
            pltpu.async_copy(table_hbm.at[idx_v], rows_v, sem).wait()
            pltpu.sync_copy(rows_v, out_hbm.at[pl.ds(off, chunk)])
            return carry

        lax.fori_loop(0, n_chunks, body, 0)

    return k(idx, weight)


def kernel(x, weight):
    b, t = x.shape
    v, d = weight.shape
    n = b * t
    out = _gather_sc(x.reshape(n), weight, n, d)
    return out.reshape(b, t, d)


# speedup vs baseline: 1.4999x; 1.0259x over previous
import functools

import jax
import jax.numpy as jnp
from jax import lax
from jax.experimental import pallas as pl
from jax.experimental.pallas import tpu as pltpu
from jax.experimental.pallas import tpu_sc as plsc


def _gather_sc(idx, weight, n, d):
    info = plsc.get_sparse_core_info()
    nc, ns = info.num_cores, info.num_subcores
    nw = nc * ns
    per_w = n // nw
    chunk = 1600
    n_chunks = per_w // chunk

    mesh = plsc.VectorSubcoreMesh(core_axis_name="c", subcore_axis_name="s")

    @functools.partial(
        pl.kernel,
        out_type=jax.ShapeDtypeStruct((n, d), jnp.float32),
        mesh=mesh,
        scratch_types=[
            pltpu.VMEM((2, chunk), jnp.int32),
            pltpu.VMEM((2, chunk, d), jnp.float32),
            pltpu.SemaphoreType.DMA((2,)),
            pltpu.SemaphoreType.DMA((2,)),
            pltpu.SemaphoreType.DMA((2,)),
        ],
        compiler_params=pltpu.CompilerParams(use_tc_tiling_on_sc=False),
    )
    def k(idx_hbm, table_hbm, out_hbm, idx_v, rows_v, isem, gsem, osem):
        wid = lax.axis_index("s") * nc + lax.axis_index("c")
        base = wid * per_w

        def load(j, b):
            return pltpu.async_copy(
                idx_hbm.at[pl.ds(base + j * chunk, chunk)], idx_v.at[b], isem.at[b]
            )

        def gather(b):
            return pltpu.async_copy(
                table_hbm.at[idx_v.at[b]], rows_v.at[b], gsem.at[b]
            )

        def write(j, b):
            return pltpu.async_copy(
                rows_v.at[b], out_hbm.at[pl.ds(base + j * chunk, chunk)], osem.at[b]
            )

        loads, g, w = {}, {}, {}
        loads[0] = load(0, 0)
        if n_chunks > 1:
            loads[1] = load(1, 1)
        loads[0].wait()
        g[0] = gather(0)
        for j in range(n_chunks):
            b = j & 1
            nb = 1 - b
            if j + 1 < n_chunks:
                loads[j + 1].wait()
                if j - 1 >= 0:
                    w[j - 1].wait()
                g[j + 1] = gather(nb)
            g[j].wait()
            w[j] = write(j, b)
            if j + 2 < n_chunks:
                loads[j + 2] = load(j + 2, b)
        if n_chunks > 1:
            w[n_chunks - 2].wait()
        w[n_chunks - 1].wait()

    return k(idx, weight)


def kernel(x, weight):
    b, t = x.shape
    v, d = weight.shape
    n = b * t
    out = _gather_sc(x.reshape(n), weight, n, d)
    return out.reshape(b, t, d)
